# baseline (speedup 1.0000x reference)
import functools

import jax
import jax.numpy as jnp
from jax import lax
from jax.experimental import pallas as pl
from jax.experimental.pallas import tpu as pltpu
from jax.experimental.pallas import tpu_sc as plsc

N = 10000
NP = 10240
E = 320000
T = 4
D = 128
NC, NS, LANES = 2, 16, 16
NW = NC * NS
EPT = E // NW
CH = 80
NCHUNK = EPT // CH
RPT = NP // NS
ZR = 32
BN = 1024

CHA = 48
CHB = CH - CHA


def _sc_gat_pass(featflat, ee, src, dst, H):
    mesh = plsc.VectorSubcoreMesh(
        core_axis_name="c", subcore_axis_name="s",
        num_cores=NC, num_subcores=NS)

    @functools.partial(
        pl.kernel,
        out_type=[jax.ShapeDtypeStruct((NC, H, NP, D), jnp.float32),
                  jax.ShapeDtypeStruct((NC * H * NP,), jnp.float32)],
        mesh=mesh,
        scratch_types=[
            pltpu.VMEM_SHARED((NP, D), jnp.float32),
            pltpu.VMEM_SHARED((NP,), jnp.float32),
            pltpu.VMEM((NP,), jnp.float32),
            pltpu.VMEM((NP,), jnp.float32),
            pltpu.VMEM((3, 1, CH), jnp.int32),
            pltpu.VMEM((3, 1, CHA), jnp.int32),
            pltpu.VMEM((3, 1, CHB), jnp.int32),
            pltpu.VMEM((CHA,), jnp.int32),
            pltpu.VMEM((CHB,), jnp.int32),
            pltpu.VMEM((CH,), jnp.float32),
            pltpu.VMEM((2, CH, D), jnp.float32),
            pltpu.VMEM((ZR, D), jnp.float32),
            pltpu.SemaphoreType.DMA,
            pltpu.SemaphoreType.DMA,
            pltpu.SemaphoreType.DMA,
            pltpu.SemaphoreType.DMA,
            pltpu.SemaphoreType.DMA,
        ],
        compiler_params=pltpu.CompilerParams(needs_layout_passes=False),
    )
    def k(feat_hbm, ee_hbm, src_hbm, dst_hbm, acc_out, den_out,
          acc_sp, den_sp, el_v, er_v, srcq2, dsta, dstb, srcga, srcgb,
          pbuf, rows, zbuf, gsem_a, gsem_b, isem, ssem_a, ssem_b):
        c = lax.axis_index("c")
        s = lax.axis_index("s")
        ebase = s * EPT + c * (NS * EPT)
        zeros16 = jnp.zeros((LANES,), jnp.float32)

        def zb_body(r, carry):
            for cc in range(D // LANES):
                zbuf[r, pl.ds(cc * LANES, LANES)] = zeros16
            return carry
        lax.fori_loop(0, ZR, zb_body, 0)

        def issue_idx(q, par):
            off = pl.multiple_of(ebase + q * CH, 16)
            pltpu.async_copy(src_hbm.at[pl.ds(off, CH)],
                             srcq2.at[par, 0], isem)
            pltpu.async_copy(dst_hbm.at[pl.ds(off, CHA)],
                             dsta.at[par, 0], isem)
            pltpu.async_copy(dst_hbm.at[pl.ds(off + CHA, CHB)],
                             dstb.at[par, 0], isem)

        def drain_idx():
            pltpu.make_async_copy(src_hbm.at[pl.ds(0, CH)],
                                  srcq2.at[0, 0], isem).wait()
            pltpu.make_async_copy(dst_hbm.at[pl.ds(0, CHA)],
                                  dsta.at[0, 0], isem).wait()
            pltpu.make_async_copy(dst_hbm.at[pl.ds(0, CHB)],
                                  dstb.at[0, 0], isem).wait()

        issue_idx(0, 0)

        for h in range(H):
            pltpu.sync_copy(ee_hbm.at[h], el_v)
            pltpu.sync_copy(ee_hbm.at[4 + h], er_v)
            for j in range(RPT // ZR):
                pltpu.sync_copy(zbuf, acc_sp.at[pl.ds(s * RPT + j * ZR, ZR)])
            for j in range(RPT // D):
                pltpu.sync_copy(zbuf.at[0],
                                den_sp.at[pl.ds(s * RPT + j * D, D)])
            plsc.subcore_barrier()

            def chunk_body(q, carry):
                par = lax.rem(q + h * NCHUNK, 3)
                nxt = lax.rem(q + 1, NCHUNK)
                parn = lax.rem(q + 1 + h * NCHUNK, 3)
                qp = lax.rem(q, 2)
                @pl.when(q >= 2)
                def _():
                    pltpu.make_async_copy(
                        feat_hbm.at[pl.ds(0, CHA)],
                        rows.at[qp, pl.ds(0, CHA)], ssem_a).wait()
                    pltpu.make_async_copy(
                        feat_hbm.at[pl.ds(0, CHB)],
                        rows.at[qp, pl.ds(CHA, CHB)], ssem_b).wait()
                drain_idx()
                for i in range(CHA // LANES):
                    sv = srcq2[par, 0, pl.ds(i * LANES, LANES)]
                    srcga[pl.ds(i * LANES, LANES)] = sv + (h * NP)
                for i in range(CHB // LANES):
                    sv = srcq2[par, 0, pl.ds(CHA + i * LANES, LANES)]
                    srcgb[pl.ds(i * LANES, LANES)] = sv + (h * NP)
                pltpu.async_copy(feat_hbm.at[srcga],
                                 rows.at[qp, pl.ds(0, CHA)], gsem_a)
                pltpu.async_copy(feat_hbm.at[srcgb],
                                 rows.at[qp, pl.ds(CHA, CHB)], gsem_b)
                issue_idx(nxt, parn)
                for i in range(CH // LANES):
                    sv = srcq2[par, 0, pl.ds(i * LANES, LANES)]
                    if i * LANES < CHA:
                        dv = dsta[par, 0, pl.ds(i * LANES, LANES)]
                    else:
                        dv = dstb[par, 0, pl.ds(i * LANES - CHA, LANES)]
                    ev = (plsc.load_gather(el_v, [sv])
                          + plsc.load_gather(er_v, [dv]))
                    ev = jnp.where(ev >= 0.0, ev, 0.2 * ev)
                    pbuf[pl.ds(i * LANES, LANES)] = jnp.exp(ev)
                pltpu.sync_copy(pbuf.at[pl.ds(0, CHA)],
                                den_sp.at[dsta.at[par, 0]], add=True)
                pltpu.sync_copy(pbuf.at[pl.ds(CHA, CHB)],
                                den_sp.at[dstb.at[par, 0]], add=True)
                def scale_rows(base, ngroups):
                    for i in range(ngroups):
                        pv = pbuf[pl.ds(base + i * LANES, LANES)]
                        bcast = [
                            pv.at[jnp.full((LANES,), j, jnp.int32)].get(
                                mode="promise_in_bounds")
                            for j in range(LANES)]
                        for cc in range(D // LANES):
                            sl = pl.ds(cc * LANES, LANES)
                            for j in range(LANES):
                                r = base + i * LANES + j
                                rows[qp, r, sl] = rows[qp, r, sl] * bcast[j]

                pltpu.make_async_copy(feat_hbm.at[srcga],
                                      rows.at[qp, pl.ds(0, CHA)],
                                      gsem_a).wait()
                scale_rows(0, CHA // LANES)
                pltpu.async_copy(rows.at[qp, pl.ds(0, CHA)],
                                 acc_sp.at[dsta.at[par, 0]], ssem_a,
                                 add=True)
                pltpu.make_async_copy(feat_hbm.at[srcgb],
                                      rows.at[qp, pl.ds(CHA, CHB)],
                                      gsem_b).wait()
                scale_rows(CHA, CHB // LANES)
                pltpu.async_copy(rows.at[qp, pl.ds(CHA, CHB)],
                                 acc_sp.at[dstb.at[par, 0]], ssem_b,
                                 add=True)
                return carry
            lax.fori_loop(0, NCHUNK, chunk_body, 0)
            for _qp in range(2):
                pltpu.make_async_copy(feat_hbm.at[pl.ds(0, CHA)],
                                      rows.at[_qp, pl.ds(0, CHA)],
                                      ssem_a).wait()
                pltpu.make_async_copy(feat_hbm.at[pl.ds(0, CHB)],
                                      rows.at[_qp, pl.ds(CHA, CHB)],
                                      ssem_b).wait()
            plsc.subcore_barrier()
            pltpu.sync_copy(acc_sp.at[pl.ds(s * RPT, RPT)],
                            acc_out.at[c, h, pl.ds(s * RPT, RPT)])
            pltpu.sync_copy(den_sp.at[pl.ds(s * RPT, RPT)],
                            den_out.at[pl.ds((c * H + h) * NP + s * RPT,
                                             RPT)])
            plsc.subcore_barrier()
        drain_idx()

    acc, denflat = k(featflat, ee, src, dst)
    return acc, denflat.reshape(NC, H, NP)


def _tc_project(x, W, attn_l_pad, attn_r_pad, H):
    Nn, Din = x.shape
    Dout = W.shape[1]

    def body(x_ref, w_ref, al_ref, ar_ref, feat_ref, ee_ref):
        xb = x_ref[...]
        f = jnp.dot(xb, w_ref[...], preferred_element_type=jnp.float32)
        el_rows, er_rows = [], []
        for h in range(H):
            fh = f[:, h * D:(h + 1) * D]
            feat_ref[h] = fh
            el_rows.append(jnp.sum(fh * al_ref[h][None, :], axis=1))
            er_rows.append(jnp.sum(fh * ar_ref[h][None, :], axis=1))
        zero = jnp.zeros((xb.shape[0],), jnp.float32)
        pad = [zero] * (4 - H)
        ee_ref[...] = jnp.stack(el_rows + pad + er_rows + pad, axis=0)

    return pl.pallas_call(
        body,
        grid=(Nn // BN,),
        in_specs=[pl.BlockSpec((BN, Din), lambda i: (i, 0)),
                  pl.BlockSpec((Din, Dout), lambda i: (0, 0)),
                  pl.BlockSpec((8, D), lambda i: (0, 0)),
                  pl.BlockSpec((8, D), lambda i: (0, 0))],
        out_specs=[pl.BlockSpec((H, BN, D), lambda i: (0, i, 0)),
                   pl.BlockSpec((8, BN), lambda i: (0, i))],
        out_shape=[jax.ShapeDtypeStruct((H, Nn, D), jnp.float32),
                   jax.ShapeDtypeStruct((8, Nn), jnp.float32)],
    )(x, W, attn_l_pad, attn_r_pad)


def _tc_combine1(acc, den, b1p):
    def body(a_ref, d_ref, b_ref, o_ref):
        for h in range(3):
            a = jnp.sum(a_ref[:, h], axis=0)
            dn = jnp.sum(d_ref[:, h, :], axis=0)
            hh = a / (dn[:, None] + 1e-9) + b_ref[h][None, :]
            o_ref[:, h * D:(h + 1) * D] = jnp.maximum(hh, 0.0)

    return pl.pallas_call(
        body,
        grid=(NP // BN,),
        in_specs=[pl.BlockSpec((NC, 3, BN, D), lambda i: (0, 0, i, 0)),
                  pl.BlockSpec((NC, 3, BN), lambda i: (0, 0, i)),
                  pl.BlockSpec((8, D), lambda i: (0, 0))],
        out_specs=pl.BlockSpec((BN, 3 * D), lambda i: (i, 0)),
        out_shape=jax.ShapeDtypeStruct((NP, 3 * D), jnp.float32),
    )(acc, den, b1p)


def _tc_final(acc, den, b2p):
    def body(a_ref, d_ref, b_ref, o_ref):
        a = jnp.sum(a_ref[:, 0], axis=0)
        dn = jnp.sum(d_ref[:, 0, :], axis=0)
        o_ref[...] = a / (dn[:, None] + 1e-9) + b_ref[0][None, :]

    return pl.pallas_call(
        body,
        grid=(NP // BN,),
        in_specs=[pl.BlockSpec((NC, 1, BN, D), lambda i: (0, 0, i, 0)),
                  pl.BlockSpec((NC, 1, BN), lambda i: (0, 0, i)),
                  pl.BlockSpec((8, D), lambda i: (0, 0))],
        out_specs=pl.BlockSpec((BN, D), lambda i: (i, 0)),
        out_shape=jax.ShapeDtypeStruct((NP, D), jnp.float32),
    )(acc, den, b2p)


def kernel(in_feat_list, edge_index, W1, attn_l1, attn_r1, b1,
           W2, attn_l2, attn_r2, b2):
    ei = edge_index.astype(jnp.int32)
    al1 = jnp.zeros((8, D), jnp.float32).at[:3].set(attn_l1)
    ar1 = jnp.zeros((8, D), jnp.float32).at[:3].set(attn_r1)
    al2 = jnp.zeros((8, D), jnp.float32).at[:1].set(attn_l2)
    ar2 = jnp.zeros((8, D), jnp.float32).at[:1].set(attn_r2)
    b1p = jnp.zeros((8, D), jnp.float32).at[:3].set(b1.reshape(3, D))
    b2p = jnp.zeros((8, D), jnp.float32).at[0].set(b2)

    xp = jnp.pad(in_feat_list, ((0, NP - N), (0, 0)))
    feat1, ee1 = _tc_project(xp, W1, al1, ar1, 3)
    featflat1 = feat1.reshape(3 * NP, D)

    outs = []
    for t in range(T):
        src = jnp.pad(ei[t, 0], (0, CH))
        dst = jnp.pad(ei[t, 1], (0, CH))
        acc1, den1 = _sc_gat_pass(featflat1, ee1, src, dst, 3)
        hfeat = _tc_combine1(acc1, den1, b1p)
        feat2, ee2 = _tc_project(hfeat, W2, al2, ar2, 1)
        acc2, den2 = _sc_gat_pass(feat2.reshape(NP, D), ee2, src, dst, 1)
        outs.append(_tc_final(acc2, den2, b2p)[:N])
    return jnp.stack(outs, axis=0)


# speedup vs baseline: 1.0519x; 1.0519x over previous
import functools

import jax
import jax.numpy as jnp
from jax import lax
from jax.experimental import pallas as pl
from jax.experimental.pallas import tpu as pltpu
from jax.experimental.pallas import tpu_sc as plsc

N = 10000
NP = 10240
E = 320000
T = 4
D = 128
NC, NS, LANES = 2, 16, 16
NW = NC * NS
EPT = E // NW
CH = 80
NCHUNK = EPT // CH
RPT = NP // NS
ZR = 32
BN = 1024

CHA = 48
CHB = CH - CHA


def _sc_gat_pass(featflat, ee, src, dst, H):
    mesh = plsc.VectorSubcoreMesh(
        core_axis_name="c", subcore_axis_name="s",
        num_cores=NC, num_subcores=NS)

    @functools.partial(
        pl.kernel,
        out_type=[jax.ShapeDtypeStruct((NC, H, NP, D), jnp.float32),
                  jax.ShapeDtypeStruct((NC * H * NP,), jnp.float32)],
        mesh=mesh,
        scratch_types=[
            pltpu.VMEM_SHARED((NP, D), jnp.float32),
            pltpu.VMEM_SHARED((NP,), jnp.float32),
            pltpu.VMEM((NP,), jnp.float32),
            pltpu.VMEM((NP,), jnp.float32),
            pltpu.VMEM((3, 1, CH), jnp.int32),
            pltpu.VMEM((3, 1, CHA), jnp.int32),
            pltpu.VMEM((3, 1, CHB), jnp.int32),
            pltpu.VMEM((CHA,), jnp.int32),
            pltpu.VMEM((CHB,), jnp.int32),
            pltpu.VMEM((CH,), jnp.float32),
            pltpu.VMEM((2, CH, D), jnp.float32),
            pltpu.VMEM((ZR, D), jnp.float32),
            pltpu.SemaphoreType.DMA,
            pltpu.SemaphoreType.DMA,
            pltpu.SemaphoreType.DMA,
            pltpu.SemaphoreType.DMA,
            pltpu.SemaphoreType.DMA,
        ],
        compiler_params=pltpu.CompilerParams(needs_layout_passes=False),
    )
    def k(feat_hbm, ee_hbm, src_hbm, dst_hbm, acc_out, den_out,
          acc_sp, den_sp, el_v, er_v, srcq2, dsta, dstb, srcga, srcgb,
          pbuf, rows, zbuf, gsem_a, gsem_b, isem, ssem_a, ssem_b):
        c = lax.axis_index("c")
        s = lax.axis_index("s")
        ebase = s * EPT + c * (NS * EPT)
        zeros16 = jnp.zeros((LANES,), jnp.float32)

        def zb_body(r, carry):
            for cc in range(D // LANES):
                zbuf[r, pl.ds(cc * LANES, LANES)] = zeros16
            return carry
        lax.fori_loop(0, ZR, zb_body, 0)

        def issue_idx(q, par):
            off = pl.multiple_of(ebase + q * CH, 16)
            pltpu.async_copy(src_hbm.at[pl.ds(off, CH)],
                             srcq2.at[par, 0], isem)
            pltpu.async_copy(dst_hbm.at[pl.ds(off, CHA)],
                             dsta.at[par, 0], isem)
            pltpu.async_copy(dst_hbm.at[pl.ds(off + CHA, CHB)],
                             dstb.at[par, 0], isem)

        def drain_idx():
            pltpu.make_async_copy(src_hbm.at[pl.ds(0, CH)],
                                  srcq2.at[0, 0], isem).wait()
            pltpu.make_async_copy(dst_hbm.at[pl.ds(0, CHA)],
                                  dsta.at[0, 0], isem).wait()
            pltpu.make_async_copy(dst_hbm.at[pl.ds(0, CHB)],
                                  dstb.at[0, 0], isem).wait()

        issue_idx(0, 0)

        for h in range(H):
            pltpu.sync_copy(ee_hbm.at[h], el_v)
            pltpu.sync_copy(ee_hbm.at[4 + h], er_v)
            for j in range(RPT // ZR):
                pltpu.sync_copy(zbuf, acc_sp.at[pl.ds(s * RPT + j * ZR, ZR)])
            for j in range(RPT // D):
                pltpu.sync_copy(zbuf.at[0],
                                den_sp.at[pl.ds(s * RPT + j * D, D)])
            plsc.subcore_barrier()

            def chunk_work(q, qp):
                par = lax.rem(q + h * NCHUNK, 3)
                nxt = lax.rem(q + 1, NCHUNK)
                parn = lax.rem(q + 1 + h * NCHUNK, 3)
                @pl.when(q >= 2)
                def _():
                    pltpu.make_async_copy(
                        feat_hbm.at[pl.ds(0, CHA)],
                        rows.at[qp, pl.ds(0, CHA)], ssem_a).wait()
                    pltpu.make_async_copy(
                        feat_hbm.at[pl.ds(0, CHB)],
                        rows.at[qp, pl.ds(CHA, CHB)], ssem_b).wait()
                drain_idx()
                for i in range(CHA // LANES):
                    sv = srcq2[par, 0, pl.ds(i * LANES, LANES)]
                    srcga[pl.ds(i * LANES, LANES)] = sv + (h * NP)
                for i in range(CHB // LANES):
                    sv = srcq2[par, 0, pl.ds(CHA + i * LANES, LANES)]
                    srcgb[pl.ds(i * LANES, LANES)] = sv + (h * NP)
                pltpu.async_copy(feat_hbm.at[srcga],
                                 rows.at[qp, pl.ds(0, CHA)], gsem_a)
                pltpu.async_copy(feat_hbm.at[srcgb],
                                 rows.at[qp, pl.ds(CHA, CHB)], gsem_b)
                issue_idx(nxt, parn)
                for i in range(CH // LANES):
                    sv = srcq2[par, 0, pl.ds(i * LANES, LANES)]
                    if i * LANES < CHA:
                        dv = dsta[par, 0, pl.ds(i * LANES, LANES)]
                    else:
                        dv = dstb[par, 0, pl.ds(i * LANES - CHA, LANES)]
                    ev = (plsc.load_gather(el_v, [sv])
                          + plsc.load_gather(er_v, [dv]))
                    ev = jnp.where(ev >= 0.0, ev, 0.2 * ev)
                    pbuf[pl.ds(i * LANES, LANES)] = jnp.exp(ev)
                pltpu.sync_copy(pbuf.at[pl.ds(0, CHA)],
                                den_sp.at[dsta.at[par, 0]], add=True)
                pltpu.sync_copy(pbuf.at[pl.ds(CHA, CHB)],
                                den_sp.at[dstb.at[par, 0]], add=True)
                def scale_rows(base, ngroups):
                    for i in range(ngroups):
                        pv = pbuf[pl.ds(base + i * LANES, LANES)]
                        bcast = [
                            pv.at[jnp.full((LANES,), j, jnp.int32)].get(
                                mode="promise_in_bounds")
                            for j in range(LANES)]
                        for cc in range(D // LANES):
                            sl = pl.ds(cc * LANES, LANES)
                            for j in range(LANES):
                                r = base + i * LANES + j
                                rows[qp, r, sl] = rows[qp, r, sl] * bcast[j]

                pltpu.make_async_copy(feat_hbm.at[srcga],
                                      rows.at[qp, pl.ds(0, CHA)],
                                      gsem_a).wait()
                scale_rows(0, CHA // LANES)
                pltpu.async_copy(rows.at[qp, pl.ds(0, CHA)],
                                 acc_sp.at[dsta.at[par, 0]], ssem_a,
                                 add=True)
                pltpu.make_async_copy(feat_hbm.at[srcgb],
                                      rows.at[qp, pl.ds(CHA, CHB)],
                                      gsem_b).wait()
                scale_rows(CHA, CHB // LANES)
                pltpu.async_copy(rows.at[qp, pl.ds(CHA, CHB)],
                                 acc_sp.at[dstb.at[par, 0]], ssem_b,
                                 add=True)

            def chunk_pair(i, carry):
                chunk_work(i * 2, 0)
                chunk_work(i * 2 + 1, 1)
                return carry
            lax.fori_loop(0, NCHUNK // 2, chunk_pair, 0)
            chunk_work(NCHUNK - 1, 0)
            for _qp in range(2):
                pltpu.make_async_copy(feat_hbm.at[pl.ds(0, CHA)],
                                      rows.at[_qp, pl.ds(0, CHA)],
                                      ssem_a).wait()
                pltpu.make_async_copy(feat_hbm.at[pl.ds(0, CHB)],
                                      rows.at[_qp, pl.ds(CHA, CHB)],
                                      ssem_b).wait()
            plsc.subcore_barrier()
            pltpu.sync_copy(acc_sp.at[pl.ds(s * RPT, RPT)],
                            acc_out.at[c, h, pl.ds(s * RPT, RPT)])
            pltpu.sync_copy(den_sp.at[pl.ds(s * RPT, RPT)],
                            den_out.at[pl.ds((c * H + h) * NP + s * RPT,
                                             RPT)])
            plsc.subcore_barrier()
        drain_idx()

    acc, denflat = k(featflat, ee, src, dst)
    return acc, denflat.reshape(NC, H, NP)


def _tc_project(x, W, attn_l_pad, attn_r_pad, H):
    Nn, Din = x.shape
    Dout = W.shape[1]

    def body(x_ref, w_ref, al_ref, ar_ref, feat_ref, ee_ref):
        xb = x_ref[...]
        f = jnp.dot(xb, w_ref[...], preferred_element_type=jnp.float32)
        el_rows, er_rows = [], []
        for h in range(H):
            fh = f[:, h * D:(h + 1) * D]
            feat_ref[h] = fh
            el_rows.append(jnp.sum(fh * al_ref[h][None, :], axis=1))
            er_rows.append(jnp.sum(fh * ar_ref[h][None, :], axis=1))
        zero = jnp.zeros((xb.shape[0],), jnp.float32)
        pad = [zero] * (4 - H)
        ee_ref[...] = jnp.stack(el_rows + pad + er_rows + pad, axis=0)

    return pl.pallas_call(
        body,
        grid=(Nn // BN,),
        in_specs=[pl.BlockSpec((BN, Din), lambda i: (i, 0)),
                  pl.BlockSpec((Din, Dout), lambda i: (0, 0)),
                  pl.BlockSpec((8, D), lambda i: (0, 0)),
                  pl.BlockSpec((8, D), lambda i: (0, 0))],
        out_specs=[pl.BlockSpec((H, BN, D), lambda i: (0, i, 0)),
                   pl.BlockSpec((8, BN), lambda i: (0, i))],
        out_shape=[jax.ShapeDtypeStruct((H, Nn, D), jnp.float32),
                   jax.ShapeDtypeStruct((8, Nn), jnp.float32)],
    )(x, W, attn_l_pad, attn_r_pad)


def _tc_combine1(acc, den, b1p):
    def body(a_ref, d_ref, b_ref, o_ref):
        for h in range(3):
            a = jnp.sum(a_ref[:, h], axis=0)
            dn = jnp.sum(d_ref[:, h, :], axis=0)
            hh = a / (dn[:, None] + 1e-9) + b_ref[h][None, :]
            o_ref[:, h * D:(h + 1) * D] = jnp.maximum(hh, 0.0)

    return pl.pallas_call(
        body,
        grid=(NP // BN,),
        in_specs=[pl.BlockSpec((NC, 3, BN, D), lambda i: (0, 0, i, 0)),
                  pl.BlockSpec((NC, 3, BN), lambda i: (0, 0, i)),
                  pl.BlockSpec((8, D), lambda i: (0, 0))],
        out_specs=pl.BlockSpec((BN, 3 * D), lambda i: (i, 0)),
        out_shape=jax.ShapeDtypeStruct((NP, 3 * D), jnp.float32),
    )(acc, den, b1p)


def _tc_final(acc, den, b2p):
    def body(a_ref, d_ref, b_ref, o_ref):
        a = jnp.sum(a_ref[:, 0], axis=0)
        dn = jnp.sum(d_ref[:, 0, :], axis=0)
        o_ref[...] = a / (dn[:, None] + 1e-9) + b_ref[0][None, :]

    return pl.pallas_call(
        body,
        grid=(NP // BN,),
        in_specs=[pl.BlockSpec((NC, 1, BN, D), lambda i: (0, 0, i, 0)),
                  pl.BlockSpec((NC, 1, BN), lambda i: (0, 0, i)),
                  pl.BlockSpec((8, D), lambda i: (0, 0))],
        out_specs=pl.BlockSpec((BN, D), lambda i: (i, 0)),
        out_shape=jax.ShapeDtypeStruct((NP, D), jnp.float32),
    )(acc, den, b2p)


def kernel(in_feat_list, edge_index, W1, attn_l1, attn_r1, b1,
           W2, attn_l2, attn_r2, b2):
    ei = edge_index.astype(jnp.int32)
    al1 = jnp.zeros((8, D), jnp.float32).at[:3].set(attn_l1)
    ar1 = jnp.zeros((8, D), jnp.float32).at[:3].set(attn_r1)
    al2 = jnp.zeros((8, D), jnp.float32).at[:1].set(attn_l2)
    ar2 = jnp.zeros((8, D), jnp.float32).at[:1].set(attn_r2)
    b1p = jnp.zeros((8, D), jnp.float32).at[:3].set(b1.reshape(3, D))
    b2p = jnp.zeros((8, D), jnp.float32).at[0].set(b2)

    xp = jnp.pad(in_feat_list, ((0, NP - N), (0, 0)))
    feat1, ee1 = _tc_project(xp, W1, al1, ar1, 3)
    featflat1 = feat1.reshape(3 * NP, D)

    outs = []
    for t in range(T):
        src = jnp.pad(ei[t, 0], (0, CH))
        dst = jnp.pad(ei[t, 1], (0, CH))
        acc1, den1 = _sc_gat_pass(featflat1, ee1, src, dst, 3)
        hfeat = _tc_combine1(acc1, den1, b1p)
        feat2, ee2 = _tc_project(hfeat, W2, al2, ar2, 1)
        acc2, den2 = _sc_gat_pass(feat2.reshape(NP, D), ee2, src, dst, 1)
        outs.append(_tc_final(acc2, den2, b2p)[:N])
    return jnp.stack(outs, axis=0)


# speedup vs baseline: 1.0851x; 1.0316x over previous
import functools

import jax
import jax.numpy as jnp
from jax import lax
from jax.experimental import pallas as pl
from jax.experimental.pallas import tpu as pltpu
from jax.experimental.pallas import tpu_sc as plsc

N = 10000
NP = 10240
E = 320000
T = 4
D = 128
NC, NS, LANES = 2, 16, 16
NW = NC * NS
EPT = E // NW
CH = 80
NCHUNK = EPT // CH
RPT = NP // NS
ZR = 32
BN = 1024

CHA = 48
CHB = CH - CHA


def _sc_gat_pass(featflat, ee, src, dst, H):
    mesh = plsc.VectorSubcoreMesh(
        core_axis_name="c", subcore_axis_name="s",
        num_cores=NC, num_subcores=NS)

    @functools.partial(
        pl.kernel,
        out_type=[jax.ShapeDtypeStruct((NC, H, NP, D), jnp.float32),
                  jax.ShapeDtypeStruct((NC * H * NP,), jnp.float32)],
        mesh=mesh,
        scratch_types=[
            pltpu.VMEM_SHARED((NP, D), jnp.float32),
            pltpu.VMEM_SHARED((NP,), jnp.float32),
            pltpu.VMEM((NP,), jnp.float32),
            pltpu.VMEM((NP,), jnp.float32),
            pltpu.VMEM((2, 1, CH), jnp.int32),
            pltpu.VMEM((2, 1, CHA), jnp.int32),
            pltpu.VMEM((2, 1, CHB), jnp.int32),
            pltpu.VMEM((CHA,), jnp.int32),
            pltpu.VMEM((CHB,), jnp.int32),
            pltpu.VMEM((CH,), jnp.float32),
            pltpu.VMEM((CH, D), jnp.float32),
            pltpu.VMEM((ZR, D), jnp.float32),
            pltpu.SemaphoreType.DMA,
            pltpu.SemaphoreType.DMA,
            pltpu.SemaphoreType.DMA,
        ],
        compiler_params=pltpu.CompilerParams(needs_layout_passes=False),
    )
    def k(feat_hbm, ee_hbm, src_hbm, dst_hbm, acc_out, den_out,
          acc_sp, den_sp, el_v, er_v, srcq2, dsta, dstb, srcga, srcgb,
          pbuf, rows, zbuf, gsem_a, gsem_b, isem):
        c = lax.axis_index("c")
        s = lax.axis_index("s")
        ebase = s * EPT + c * (NS * EPT)
        zeros16 = jnp.zeros((LANES,), jnp.float32)

        def zb_body(r, carry):
            for cc in range(D // LANES):
                zbuf[r, pl.ds(cc * LANES, LANES)] = zeros16
            return carry
        lax.fori_loop(0, ZR, zb_body, 0)

        def issue_idx(q, par):
            off = pl.multiple_of(ebase + q * CH, 16)
            pltpu.async_copy(src_hbm.at[pl.ds(off, CH)],
                             srcq2.at[par, 0], isem)
            pltpu.async_copy(dst_hbm.at[pl.ds(off, CHA)],
                             dsta.at[par, 0], isem)
            pltpu.async_copy(dst_hbm.at[pl.ds(off + CHA, CHB)],
                             dstb.at[par, 0], isem)

        def drain_idx():
            pltpu.make_async_copy(src_hbm.at[pl.ds(0, CH)],
                                  srcq2.at[0, 0], isem).wait()
            pltpu.make_async_copy(dst_hbm.at[pl.ds(0, CHA)],
                                  dsta.at[0, 0], isem).wait()
            pltpu.make_async_copy(dst_hbm.at[pl.ds(0, CHB)],
                                  dstb.at[0, 0], isem).wait()

        issue_idx(0, 0)

        for h in range(H):
            pltpu.sync_copy(ee_hbm.at[h], el_v)
            pltpu.sync_copy(ee_hbm.at[4 + h], er_v)
            for j in range(RPT // ZR):
                pltpu.sync_copy(zbuf, acc_sp.at[pl.ds(s * RPT + j * ZR, ZR)])
            for j in range(RPT // D):
                pltpu.sync_copy(zbuf.at[0],
                                den_sp.at[pl.ds(s * RPT + j * D, D)])
            plsc.subcore_barrier()

            def chunk_body(q, carry):
                par = lax.rem(q + h * NCHUNK, 2)
                nxt = lax.rem(q + 1, NCHUNK)
                parn = lax.rem(q + 1 + h * NCHUNK, 2)
                drain_idx()
                for i in range(CHA // LANES):
                    sv = srcq2[par, 0, pl.ds(i * LANES, LANES)]
                    srcga[pl.ds(i * LANES, LANES)] = sv + (h * NP)
                for i in range(CHB // LANES):
                    sv = srcq2[par, 0, pl.ds(CHA + i * LANES, LANES)]
                    srcgb[pl.ds(i * LANES, LANES)] = sv + (h * NP)
                pltpu.async_copy(feat_hbm.at[srcga],
                                 rows.at[pl.ds(0, CHA)], gsem_a)
                pltpu.async_copy(feat_hbm.at[srcgb],
                                 rows.at[pl.ds(CHA, CHB)], gsem_b)
                issue_idx(nxt, parn)
                for i in range(CH // LANES):
                    sv = srcq2[par, 0, pl.ds(i * LANES, LANES)]
                    if i * LANES < CHA:
                        dv = dsta[par, 0, pl.ds(i * LANES, LANES)]
                    else:
                        dv = dstb[par, 0, pl.ds(i * LANES - CHA, LANES)]
                    ev = (plsc.load_gather(el_v, [sv])
                          + plsc.load_gather(er_v, [dv]))
                    ev = jnp.where(ev >= 0.0, ev, 0.2 * ev)
                    pbuf[pl.ds(i * LANES, LANES)] = jnp.exp(ev)
                pltpu.sync_copy(pbuf.at[pl.ds(0, CHA)],
                                den_sp.at[dsta.at[par, 0]], add=True)
                pltpu.sync_copy(pbuf.at[pl.ds(CHA, CHB)],
                                den_sp.at[dstb.at[par, 0]], add=True)
                def scale_rows(base, ngroups):
                    for i in range(ngroups):
                        pv = pbuf[pl.ds(base + i * LANES, LANES)]
                        bcast = [
                            pv.at[jnp.full((LANES,), j, jnp.int32)].get(
                                mode="promise_in_bounds")
                            for j in range(LANES)]
                        for cc in range(D // LANES):
                            sl = pl.ds(cc * LANES, LANES)
                            for j in range(LANES):
                                r = base + i * LANES + j
                                rows[r, sl] = rows[r, sl] * bcast[j]

                pltpu.make_async_copy(feat_hbm.at[srcga],
                                      rows.at[pl.ds(0, CHA)], gsem_a).wait()
                scale_rows(0, CHA // LANES)
                pltpu.sync_copy(rows.at[pl.ds(0, CHA)],
                                acc_sp.at[dsta.at[par, 0]], add=True)
                pltpu.make_async_copy(feat_hbm.at[srcgb],
                                      rows.at[pl.ds(CHA, CHB)],
                                      gsem_b).wait()
                scale_rows(CHA, CHB // LANES)
                pltpu.sync_copy(rows.at[pl.ds(CHA, CHB)],
                                acc_sp.at[dstb.at[par, 0]], add=True)
                return carry
            lax.fori_loop(0, NCHUNK, chunk_body, 0)
            plsc.subcore_barrier()
            pltpu.sync_copy(acc_sp.at[pl.ds(s * RPT, RPT)],
                            acc_out.at[c, h, pl.ds(s * RPT, RPT)])
            pltpu.sync_copy(den_sp.at[pl.ds(s * RPT, RPT)],
                            den_out.at[pl.ds((c * H + h) * NP + s * RPT,
                                             RPT)])
            plsc.subcore_barrier()
        drain_idx()

    acc, denflat = k(featflat, ee, src, dst)
    return acc, denflat.reshape(NC, H, NP)


def _tc_project(x, W, attn_l_pad, attn_r_pad, H):
    Nn, Din = x.shape
    Dout = W.shape[1]

    def body(x_ref, w_ref, al_ref, ar_ref, feat_ref, ee_ref):
        xb = x_ref[...]
        f = jnp.dot(xb, w_ref[...], preferred_element_type=jnp.float32)
        el_rows, er_rows = [], []
        for h in range(H):
            fh = f[:, h * D:(h + 1) * D]
            feat_ref[h] = fh
            el_rows.append(jnp.sum(fh * al_ref[h][None, :], axis=1))
            er_rows.append(jnp.sum(fh * ar_ref[h][None, :], axis=1))
        zero = jnp.zeros((xb.shape[0],), jnp.float32)
        pad = [zero] * (4 - H)
        ee_ref[...] = jnp.stack(el_rows + pad + er_rows + pad, axis=0)

    return pl.pallas_call(
        body,
        grid=(Nn // BN,),
        in_specs=[pl.BlockSpec((BN, Din), lambda i: (i, 0)),
                  pl.BlockSpec((Din, Dout), lambda i: (0, 0)),
                  pl.BlockSpec((8, D), lambda i: (0, 0)),
                  pl.BlockSpec((8, D), lambda i: (0, 0))],
        out_specs=[pl.BlockSpec((H, BN, D), lambda i: (0, i, 0)),
                   pl.BlockSpec((8, BN), lambda i: (0, i))],
        out_shape=[jax.ShapeDtypeStruct((H, Nn, D), jnp.float32),
                   jax.ShapeDtypeStruct((8, Nn), jnp.float32)],
    )(x, W, attn_l_pad, attn_r_pad)


def _tc_combine1(acc, den, b1p):
    def body(a_ref, d_ref, b_ref, o_ref):
        for h in range(3):
            a = jnp.sum(a_ref[:, h], axis=0)
            dn = jnp.sum(d_ref[:, h, :], axis=0)
            hh = a / (dn[:, None] + 1e-9) + b_ref[h][None, :]
            o_ref[:, h * D:(h + 1) * D] = jnp.maximum(hh, 0.0)

    return pl.pallas_call(
        body,
        grid=(NP // BN,),
        in_specs=[pl.BlockSpec((NC, 3, BN, D), lambda i: (0, 0, i, 0)),
                  pl.BlockSpec((NC, 3, BN), lambda i: (0, 0, i)),
                  pl.BlockSpec((8, D), lambda i: (0, 0))],
        out_specs=pl.BlockSpec((BN, 3 * D), lambda i: (i, 0)),
        out_shape=jax.ShapeDtypeStruct((NP, 3 * D), jnp.float32),
    )(acc, den, b1p)


def _tc_final(acc, den, b2p):
    def body(a_ref, d_ref, b_ref, o_ref):
        a = jnp.sum(a_ref[:, 0], axis=0)
        dn = jnp.sum(d_ref[:, 0, :], axis=0)
        o_ref[...] = a / (dn[:, None] + 1e-9) + b_ref[0][None, :]

    return pl.pallas_call(
        body,
        grid=(NP // BN,),
        in_specs=[pl.BlockSpec((NC, 1, BN, D), lambda i: (0, 0, i, 0)),
                  pl.BlockSpec((NC, 1, BN), lambda i: (0, 0, i)),
                  pl.BlockSpec((8, D), lambda i: (0, 0))],
        out_specs=pl.BlockSpec((BN, D), lambda i: (i, 0)),
        out_shape=jax.ShapeDtypeStruct((NP, D), jnp.float32),
    )(acc, den, b2p)


def kernel(in_feat_list, edge_index, W1, attn_l1, attn_r1, b1,
           W2, attn_l2, attn_r2, b2):
    ei = edge_index.astype(jnp.int32)
    al1 = jnp.zeros((8, D), jnp.float32).at[:3].set(attn_l1)
    ar1 = jnp.zeros((8, D), jnp.float32).at[:3].set(attn_r1)
    al2 = jnp.zeros((8, D), jnp.float32).at[:1].set(attn_l2)
    ar2 = jnp.zeros((8, D), jnp.float32).at[:1].set(attn_r2)
    b1p = jnp.zeros((8, D), jnp.float32).at[:3].set(b1.reshape(3, D))
    b2p = jnp.zeros((8, D), jnp.float32).at[0].set(b2)

    xp = jnp.pad(in_feat_list, ((0, NP - N), (0, 0)))
    feat1, ee1 = _tc_project(xp, W1, al1, ar1, 3)
    featflat1 = feat1.reshape(3 * NP, D)

    outs = []
    for t in range(T):
        src = jnp.pad(ei[t, 0], (0, CH))
        dst = jnp.pad(ei[t, 1], (0, CH))
        acc1, den1 = _sc_gat_pass(featflat1, ee1, src, dst, 3)
        hfeat = _tc_combine1(acc1, den1, b1p)
        feat2, ee2 = _tc_project(hfeat, W2, al2, ar2, 1)
        acc2, den2 = _sc_gat_pass(feat2.reshape(NP, D), ee2, src, dst, 1)
        outs.append(_tc_final(acc2, den2, b2p)[:N])
    return jnp.stack(outs, axis=0)


# speedup vs baseline: 1.1195x; 1.0317x over previous
import functools

import jax
import jax.numpy as jnp
from jax import lax
from jax.experimental import pallas as pl
from jax.experimental.pallas import tpu as pltpu
from jax.experimental.pallas import tpu_sc as plsc

N = 10000
NP = 10240
E = 320000
T = 4
D = 128
NC, NS, LANES = 2, 16, 16
NW = NC * NS
EPT = E // NW
CH = 80
NCHUNK = EPT // CH
RPT = NP // NS
ZR = 32
BN = 1024

CHA = 48
CHB = CH - CHA


def _sc_gat_pass(featflat, ee, src, dst, H):
    mesh = plsc.VectorSubcoreMesh(
        core_axis_name="c", subcore_axis_name="s",
        num_cores=NC, num_subcores=NS)

    @functools.partial(
        pl.kernel,
        out_type=[jax.ShapeDtypeStruct((NC, H, NP, D), jnp.float32),
                  jax.ShapeDtypeStruct((NC * H * NP,), jnp.float32)],
        mesh=mesh,
        scratch_types=[
            pltpu.VMEM_SHARED((NP, D), jnp.float32),
            pltpu.VMEM_SHARED((NP,), jnp.float32),
            pltpu.VMEM((NP,), jnp.float32),
            pltpu.VMEM((NP,), jnp.float32),
            pltpu.VMEM((2, 1, CH), jnp.int32),
            pltpu.VMEM((2, 1, CH), jnp.int32),
            pltpu.VMEM((CHA,), jnp.int32),
            pltpu.VMEM((CHB,), jnp.int32),
            pltpu.VMEM((CH,), jnp.float32),
            pltpu.VMEM((CH, D), jnp.float32),
            pltpu.VMEM((ZR, D), jnp.float32),
            pltpu.SemaphoreType.DMA,
            pltpu.SemaphoreType.DMA,
            pltpu.SemaphoreType.DMA,
        ],
        compiler_params=pltpu.CompilerParams(needs_layout_passes=False),
    )
    def k(feat_hbm, ee_hbm, src_hbm, dst_hbm, acc_out, den_out,
          acc_sp, den_sp, el_v, er_v, srcq2, dstq2, srcga, srcgb,
          pbuf, rows, zbuf, gsem_a, gsem_b, isem):
        c = lax.axis_index("c")
        s = lax.axis_index("s")
        ebase = s * EPT + c * (NS * EPT)
        zeros16 = jnp.zeros((LANES,), jnp.float32)

        def zb_body(r, carry):
            for cc in range(D // LANES):
                zbuf[r, pl.ds(cc * LANES, LANES)] = zeros16
            return carry
        lax.fori_loop(0, ZR, zb_body, 0)

        def issue_idx(q, par):
            off = pl.multiple_of(ebase + q * CH, 16)
            pltpu.async_copy(src_hbm.at[pl.ds(off, CH)],
                             srcq2.at[par, 0], isem)
            pltpu.async_copy(dst_hbm.at[pl.ds(off, CH)],
                             dstq2.at[par, 0], isem)

        def drain_idx():
            pltpu.make_async_copy(src_hbm.at[pl.ds(0, CH)],
                                  srcq2.at[0, 0], isem).wait()
            pltpu.make_async_copy(dst_hbm.at[pl.ds(0, CH)],
                                  dstq2.at[0, 0], isem).wait()

        issue_idx(0, 0)

        for h in range(H):
            pltpu.sync_copy(ee_hbm.at[h], el_v)
            pltpu.sync_copy(ee_hbm.at[4 + h], er_v)
            for j in range(RPT // ZR):
                pltpu.sync_copy(zbuf, acc_sp.at[pl.ds(s * RPT + j * ZR, ZR)])
            for j in range(RPT // D):
                pltpu.sync_copy(zbuf.at[0],
                                den_sp.at[pl.ds(s * RPT + j * D, D)])
            plsc.subcore_barrier()

            def chunk_body(q, carry):
                par = lax.rem(q + h * NCHUNK, 2)
                nxt = lax.rem(q + 1, NCHUNK)
                parn = lax.rem(q + 1 + h * NCHUNK, 2)
                drain_idx()
                for i in range(CHA // LANES):
                    sv = srcq2[par, 0, pl.ds(i * LANES, LANES)]
                    srcga[pl.ds(i * LANES, LANES)] = sv + (h * NP)
                for i in range(CHB // LANES):
                    sv = srcq2[par, 0, pl.ds(CHA + i * LANES, LANES)]
                    srcgb[pl.ds(i * LANES, LANES)] = sv + (h * NP)
                pltpu.async_copy(feat_hbm.at[srcga],
                                 rows.at[pl.ds(0, CHA)], gsem_a)
                pltpu.async_copy(feat_hbm.at[srcgb],
                                 rows.at[pl.ds(CHA, CHB)], gsem_b)
                issue_idx(nxt, parn)
                for i in range(CH // LANES):
                    sv = srcq2[par, 0, pl.ds(i * LANES, LANES)]
                    dv = dstq2[par, 0, pl.ds(i * LANES, LANES)]
                    ev = (plsc.load_gather(el_v, [sv])
                          + plsc.load_gather(er_v, [dv]))
                    ev = jnp.where(ev >= 0.0, ev, 0.2 * ev)
                    pbuf[pl.ds(i * LANES, LANES)] = jnp.exp(ev)
                pltpu.sync_copy(pbuf, den_sp.at[dstq2.at[par, 0]],
                                add=True)
                def scale_rows(base, ngroups):
                    for i in range(ngroups):
                        pv = pbuf[pl.ds(base + i * LANES, LANES)]
                        bcast = [
                            pv.at[jnp.full((LANES,), j, jnp.int32)].get(
                                mode="promise_in_bounds")
                            for j in range(LANES)]
                        for cc in range(D // LANES):
                            sl = pl.ds(cc * LANES, LANES)
                            for j in range(LANES):
                                r = base + i * LANES + j
                                rows[r, sl] = rows[r, sl] * bcast[j]

                pltpu.make_async_copy(feat_hbm.at[srcga],
                                      rows.at[pl.ds(0, CHA)], gsem_a).wait()
                scale_rows(0, CHA // LANES)
                pltpu.make_async_copy(feat_hbm.at[srcgb],
                                      rows.at[pl.ds(CHA, CHB)],
                                      gsem_b).wait()
                scale_rows(CHA, CHB // LANES)
                pltpu.sync_copy(rows, acc_sp.at[dstq2.at[par, 0]],
                                add=True)
                return carry
            lax.fori_loop(0, NCHUNK, chunk_body, 0)
            plsc.subcore_barrier()
            pltpu.sync_copy(acc_sp.at[pl.ds(s * RPT, RPT)],
                            acc_out.at[c, h, pl.ds(s * RPT, RPT)])
            pltpu.sync_copy(den_sp.at[pl.ds(s * RPT, RPT)],
                            den_out.at[pl.ds((c * H + h) * NP + s * RPT,
                                             RPT)])
            plsc.subcore_barrier()
        drain_idx()

    acc, denflat = k(featflat, ee, src, dst)
    return acc, denflat.reshape(NC, H, NP)


def _tc_project(x, W, attn_l_pad, attn_r_pad, H):
    Nn, Din = x.shape
    Dout = W.shape[1]

    def body(x_ref, w_ref, al_ref, ar_ref, feat_ref, ee_ref):
        xb = x_ref[...]
        f = jnp.dot(xb, w_ref[...], preferred_element_type=jnp.float32)
        el_rows, er_rows = [], []
        for h in range(H):
            fh = f[:, h * D:(h + 1) * D]
            feat_ref[h] = fh
            el_rows.append(jnp.sum(fh * al_ref[h][None, :], axis=1))
            er_rows.append(jnp.sum(fh * ar_ref[h][None, :], axis=1))
        zero = jnp.zeros((xb.shape[0],), jnp.float32)
        pad = [zero] * (4 - H)
        ee_ref[...] = jnp.stack(el_rows + pad + er_rows + pad, axis=0)

    return pl.pallas_call(
        body,
        grid=(Nn // BN,),
        in_specs=[pl.BlockSpec((BN, Din), lambda i: (i, 0)),
                  pl.BlockSpec((Din, Dout), lambda i: (0, 0)),
                  pl.BlockSpec((8, D), lambda i: (0, 0)),
                  pl.BlockSpec((8, D), lambda i: (0, 0))],
        out_specs=[pl.BlockSpec((H, BN, D), lambda i: (0, i, 0)),
                   pl.BlockSpec((8, BN), lambda i: (0, i))],
        out_shape=[jax.ShapeDtypeStruct((H, Nn, D), jnp.float32),
                   jax.ShapeDtypeStruct((8, Nn), jnp.float32)],
    )(x, W, attn_l_pad, attn_r_pad)


def _tc_combine1(acc, den, b1p):
    def body(a_ref, d_ref, b_ref, o_ref):
        for h in range(3):
            a = jnp.sum(a_ref[:, h], axis=0)
            dn = jnp.sum(d_ref[:, h, :], axis=0)
            hh = a / (dn[:, None] + 1e-9) + b_ref[h][None, :]
            o_ref[:, h * D:(h + 1) * D] = jnp.maximum(hh, 0.0)

    return pl.pallas_call(
        body,
        grid=(NP // BN,),
        in_specs=[pl.BlockSpec((NC, 3, BN, D), lambda i: (0, 0, i, 0)),
                  pl.BlockSpec((NC, 3, BN), lambda i: (0, 0, i)),
                  pl.BlockSpec((8, D), lambda i: (0, 0))],
        out_specs=pl.BlockSpec((BN, 3 * D), lambda i: (i, 0)),
        out_shape=jax.ShapeDtypeStruct((NP, 3 * D), jnp.float32),
    )(acc, den, b1p)


def _tc_final(acc, den, b2p):
    def body(a_ref, d_ref, b_ref, o_ref):
        a = jnp.sum(a_ref[:, 0], axis=0)
        dn = jnp.sum(d_ref[:, 0, :], axis=0)
        o_ref[...] = a / (dn[:, None] + 1e-9) + b_ref[0][None, :]

    return pl.pallas_call(
        body,
        grid=(NP // BN,),
        in_specs=[pl.BlockSpec((NC, 1, BN, D), lambda i: (0, 0, i, 0)),
                  pl.BlockSpec((NC, 1, BN), lambda i: (0, 0, i)),
                  pl.BlockSpec((8, D), lambda i: (0, 0))],
        out_specs=pl.BlockSpec((BN, D), lambda i: (i, 0)),
        out_shape=jax.ShapeDtypeStruct((NP, D), jnp.float32),
    )(acc, den, b2p)


def kernel(in_feat_list, edge_index, W1, attn_l1, attn_r1, b1,
           W2, attn_l2, attn_r2, b2):
    ei = edge_index.astype(jnp.int32)
    al1 = jnp.zeros((8, D), jnp.float32).at[:3].set(attn_l1)
    ar1 = jnp.zeros((8, D), jnp.float32).at[:3].set(attn_r1)
    al2 = jnp.zeros((8, D), jnp.float32).at[:1].set(attn_l2)
    ar2 = jnp.zeros((8, D), jnp.float32).at[:1].set(attn_r2)
    b1p = jnp.zeros((8, D), jnp.float32).at[:3].set(b1.reshape(3, D))
    b2p = jnp.zeros((8, D), jnp.float32).at[0].set(b2)

    xp = jnp.pad(in_feat_list, ((0, NP - N), (0, 0)))
    feat1, ee1 = _tc_project(xp, W1, al1, ar1, 3)
    featflat1 = feat1.reshape(3 * NP, D)

    outs = []
    for t in range(T):
        src = jnp.pad(ei[t, 0], (0, CH))
        dst = jnp.pad(ei[t, 1], (0, CH))
        acc1, den1 = _sc_gat_pass(featflat1, ee1, src, dst, 3)
        hfeat = _tc_combine1(acc1, den1, b1p)
        feat2, ee2 = _tc_project(hfeat, W2, al2, ar2, 1)
        acc2, den2 = _sc_gat_pass(feat2.reshape(NP, D), ee2, src, dst, 1)
        outs.append(_tc_final(acc2, den2, b2p)[:N])
    return jnp.stack(outs, axis=0)
